# baseline (speedup 1.0000x reference)
import jax
import jax.numpy as jnp
from jax.experimental import pallas as pl


def _tc_body(xq_ref, ek_ref, epk_ref, epv_ref, ek_out, ev_out):
    xq = xq_ref[...]
    ek = ek_ref[...]
    nk = ek / jnp.maximum(
        jnp.sqrt(jnp.sum(ek * ek, axis=1, keepdims=True)), 1e-12)
    nq = xq / jnp.maximum(
        jnp.sqrt(jnp.sum(xq * xq, axis=1, keepdims=True)), 1e-12)
    scores = jax.lax.dot_general(nq, nk, (((1,), (1,)), ((), ())))
    idx = jnp.argmax(scores, axis=1)
    onehot = (jax.lax.broadcasted_iota(jnp.int32, scores.shape, 1)
              == idx[:, None]).astype(jnp.float32)
    hi = jax.lax.Precision.HIGHEST
    ek_out[...] = jax.lax.dot_general(
        onehot, epk_ref[...], (((1,), (0,)), ((), ())), precision=hi)
    ev_out[...] = jax.lax.dot_general(
        onehot, epv_ref[...], (((1,), (0,)), ((), ())), precision=hi)


def _impl(x_querry, x_block, e_p, e_k, interpret=False):
    b, key_d = x_querry.shape
    pool, p_len, emb_d = e_p.shape
    half = p_len // 2
    hd = half * emb_d
    blk = 512
    grid = (b // blk,)

    epk = e_p[:, :half, :].reshape(pool, hd)
    epv = e_p[:, half:, :].reshape(pool, hd)

    ekf, evf = pl.pallas_call(
        _tc_body,
        grid=grid,
        in_specs=[
            pl.BlockSpec((blk, key_d), lambda i: (i, 0)),
            pl.BlockSpec((pool, key_d), lambda i: (0, 0)),
            pl.BlockSpec((pool, hd), lambda i: (0, 0)),
            pl.BlockSpec((pool, hd), lambda i: (0, 0)),
        ],
        out_specs=[
            pl.BlockSpec((blk, hd), lambda i: (i, 0)),
            pl.BlockSpec((blk, hd), lambda i: (i, 0)),
        ],
        out_shape=[
            jax.ShapeDtypeStruct((b, hd), jnp.float32),
            jax.ShapeDtypeStruct((b, hd), jnp.float32),
        ],
        interpret=interpret,
    )(x_querry, e_k, epk, epv)
    return (ekf.reshape(b, half, emb_d), evf.reshape(b, half, emb_d))


def kernel(x_querry, l, x_block, e_p, e_k):
    ek_out, ev_out = _impl(x_querry, x_block, e_p, e_k)
    return (ek_out, ev_out, x_block)
